# baseline (speedup 1.0000x reference)
import functools

import jax
import jax.numpy as jnp
from jax.experimental import pallas as pl
from jax.experimental.pallas import tpu as pltpu

_BAND = 128


def _round_up(x, m):
    return (x + m - 1) // m * m


def _fm_kernel(idx_t_ref, lhs_t_ref, out_ref, *, emb_dim, n_fields):
    _, n_blk = idx_t_ref.shape
    d_pad, _ = lhs_t_ref.shape

    idx_all = idx_t_ref[...]
    iota_b = jax.lax.broadcasted_iota(jnp.int32, (_BAND, n_blk), 0)

    counts = jnp.concatenate(
        [(iota_b == idx_all[f : f + 1, :]).astype(jnp.float32)
         for f in range(n_fields)], axis=0)

    res = jnp.dot(lhs_t_ref[...], counts,
                  preferred_element_type=jnp.float32)

    row_id = jax.lax.broadcasted_iota(jnp.int32, (d_pad, n_blk), 0)
    contrib = jnp.where(row_id < emb_dim, res * res, res)
    logit = jnp.sum(contrib, axis=0, keepdims=True)
    out_ref[...] = jax.nn.sigmoid(logit)


def _lhs_kernel(scal_ref, table_ref, w_ref, lhs_ref, fused_ref, *, emb_dim,
                n_fields, d_pad):
    V, D = table_ref.shape
    tab = table_ref[...]
    fused_ref[:V, :D] = tab
    fused_ref[:V, D + 1:] = jnp.zeros((V, 128 - D - 1), jnp.float32)
    fused_ref[:V, D:D + 1] = w_ref[...] - jnp.sum(tab * tab, axis=1,
                                                  keepdims=True)
    iota_s = jax.lax.broadcasted_iota(jnp.int32, (_BAND, 128), 0)
    for f in range(n_fields):
        off = scal_ref[0, f]
        ffv = scal_ref[1, f]
        base = pl.multiple_of((off >> 3) << 3, 8)
        chunk = fused_ref[pl.ds(base, 2 * _BAND), :]
        band = pltpu.roll(chunk, -(off & 7), axis=0)[:_BAND, :]
        band = jnp.where(iota_s < ffv, band, 0.0)
        lhs_ref[:, f * _BAND:(f + 1) * _BAND] = band.T[:d_pad, :]


def _build_lhs(table, w, offsets, field_features, d_pad):
    V, D = table.shape
    F = offsets.shape[0]
    scal = jnp.stack([offsets, jnp.asarray(field_features, jnp.int32)])
    grid_spec = pltpu.PrefetchScalarGridSpec(
        num_scalar_prefetch=1,
        grid=(1,),
        in_specs=[
            pl.BlockSpec((V, D), lambda i, s: (0, 0)),
            pl.BlockSpec((V, 1), lambda i, s: (0, 0)),
        ],
        out_specs=pl.BlockSpec((d_pad, F * _BAND), lambda i, s: (0, 0)),
        scratch_shapes=[pltpu.VMEM((_round_up(V + 2 * _BAND, 8), 128),
                                   jnp.float32)],
    )
    return pl.pallas_call(
        functools.partial(_lhs_kernel, emb_dim=D, n_fields=F, d_pad=d_pad),
        out_shape=jax.ShapeDtypeStruct((d_pad, F * _BAND), jnp.float32),
        grid_spec=grid_spec,
        compiler_params=pltpu.CompilerParams(
            dimension_semantics=("arbitrary",)),
    )(scal, table, w)


def _fm_forward(x_raw, field_features, table, w, *, n_blk=4096):
    N, F = x_raw.shape
    V, D = table.shape

    n_pad = _round_up(N, n_blk)
    d_rows = D + 1
    d_pad = _round_up(d_rows, 8)
    v_loc = F * _BAND

    x_in = x_raw.astype(jnp.int32)
    if n_pad != N:
        x_in = jnp.pad(x_in, ((0, n_pad - N), (0, 0)))
    idx_t = x_in.T

    offsets = jnp.concatenate(
        [jnp.zeros((1,), jnp.int32),
         jnp.cumsum(jnp.asarray(field_features, jnp.int32))[:-1]])
    lhs_t = _build_lhs(table, w, offsets, field_features, d_pad)

    vmem_bytes = int(
        4 * (2 * d_pad * v_loc
             + 2 * F * n_blk
             + 2 * 1 * n_blk
             + 2 * d_pad * n_blk)
        + (24 << 20))

    out = pl.pallas_call(
        functools.partial(_fm_kernel, emb_dim=D, n_fields=F),
        out_shape=jax.ShapeDtypeStruct((1, n_pad), jnp.float32),
        grid=(n_pad // n_blk,),
        in_specs=[
            pl.BlockSpec((F, n_blk), lambda i: (0, i)),
            pl.BlockSpec((d_pad, v_loc), lambda i: (0, 0)),
        ],
        out_specs=pl.BlockSpec((1, n_blk), lambda i: (0, i)),
        compiler_params=pltpu.CompilerParams(
            dimension_semantics=("parallel",),
            vmem_limit_bytes=vmem_bytes),
    )(idx_t, lhs_t)

    return out[0, :N].reshape(N, 1)


def kernel(x_raw, field_features, table, w):
    return _fm_forward(x_raw, field_features, table, w)


# speedup vs baseline: 1.0329x; 1.0329x over previous
import functools

import jax
import jax.numpy as jnp
from jax.experimental import pallas as pl
from jax.experimental.pallas import tpu as pltpu

_BAND = 128


def _round_up(x, m):
    return (x + m - 1) // m * m


def _fm_kernel(idx_t_ref, lhs_t_ref, out_ref, *, emb_dim, n_fields):
    _, n_blk = idx_t_ref.shape
    d_pad, _ = lhs_t_ref.shape

    idx_all = idx_t_ref[...]
    iota_b = jax.lax.broadcasted_iota(jnp.int32, (_BAND, n_blk), 0)

    counts = jnp.concatenate(
        [(iota_b == idx_all[f : f + 1, :]).astype(jnp.float32)
         for f in range(n_fields)], axis=0)

    res = jnp.dot(lhs_t_ref[...], counts,
                  preferred_element_type=jnp.float32)

    row_id = jax.lax.broadcasted_iota(jnp.int32, (d_pad, n_blk), 0)
    contrib = jnp.where(row_id < emb_dim, res * res, res)
    logit = jnp.sum(contrib, axis=0, keepdims=True)
    out_ref[...] = jax.nn.sigmoid(logit)


def _lhs_kernel(scal_ref, table_ref, w_ref, lhs_ref, fused_ref, *, emb_dim,
                n_fields, d_pad):
    V, D = table_ref.shape
    tab = table_ref[...]
    fused_ref[:V, :D] = tab
    fused_ref[:V, D + 1:] = jnp.zeros((V, 128 - D - 1), jnp.float32)
    fused_ref[:V, D:D + 1] = w_ref[...] - jnp.sum(tab * tab, axis=1,
                                                  keepdims=True)
    iota_s = jax.lax.broadcasted_iota(jnp.int32, (_BAND, 128), 0)
    for f in range(n_fields):
        off = scal_ref[0, f]
        ffv = scal_ref[1, f]
        base = pl.multiple_of((off >> 3) << 3, 8)
        chunk = fused_ref[pl.ds(base, 2 * _BAND), :]
        band = pltpu.roll(chunk, -(off & 7), axis=0)[:_BAND, :]
        band = jnp.where(iota_s < ffv, band, 0.0)
        lhs_ref[:, f * _BAND:(f + 1) * _BAND] = band.T[:d_pad, :]


def _build_lhs(table, w, offsets, field_features, d_pad):
    V, D = table.shape
    F = offsets.shape[0]
    scal = jnp.stack([offsets, jnp.asarray(field_features, jnp.int32)])
    grid_spec = pltpu.PrefetchScalarGridSpec(
        num_scalar_prefetch=1,
        grid=(1,),
        in_specs=[
            pl.BlockSpec((V, D), lambda i, s: (0, 0)),
            pl.BlockSpec((V, 1), lambda i, s: (0, 0)),
        ],
        out_specs=pl.BlockSpec((d_pad, F * _BAND), lambda i, s: (0, 0)),
        scratch_shapes=[pltpu.VMEM((_round_up(V + 2 * _BAND, 8), 128),
                                   jnp.float32)],
    )
    return pl.pallas_call(
        functools.partial(_lhs_kernel, emb_dim=D, n_fields=F, d_pad=d_pad),
        out_shape=jax.ShapeDtypeStruct((d_pad, F * _BAND), jnp.float32),
        grid_spec=grid_spec,
        compiler_params=pltpu.CompilerParams(
            dimension_semantics=("arbitrary",)),
    )(scal, table, w)


def _fm_forward(x_raw, field_features, table, w, *, n_blk=4096):
    N, F = x_raw.shape
    V, D = table.shape

    n_pad = _round_up(N, n_blk)
    d_rows = D + 1
    d_pad = _round_up(d_rows, 8)
    v_loc = F * _BAND

    x_in = x_raw.astype(jnp.int32)
    if n_pad != N:
        x_in = jnp.pad(x_in, ((0, n_pad - N), (0, 0)))
    idx_t = x_in.T

    offsets = jnp.concatenate(
        [jnp.zeros((1,), jnp.int32),
         jnp.cumsum(jnp.asarray(field_features, jnp.int32))[:-1]])
    lhs_t = _build_lhs(table, w, offsets, field_features, d_pad)

    vmem_bytes = int(
        4 * (2 * d_pad * v_loc
             + 2 * F * n_blk
             + 2 * 1 * n_blk
             + v_loc * n_blk
             + 2 * d_pad * n_blk)
        + (8 << 20))

    out = pl.pallas_call(
        functools.partial(_fm_kernel, emb_dim=D, n_fields=F),
        out_shape=jax.ShapeDtypeStruct((1, n_pad), jnp.float32),
        grid=(n_pad // n_blk,),
        in_specs=[
            pl.BlockSpec((F, n_blk), lambda i: (0, i)),
            pl.BlockSpec((d_pad, v_loc), lambda i: (0, 0)),
        ],
        out_specs=pl.BlockSpec((1, n_blk), lambda i: (0, i)),
        compiler_params=pltpu.CompilerParams(
            dimension_semantics=("parallel",),
            vmem_limit_bytes=vmem_bytes),
    )(idx_t, lhs_t)

    return out[0, :N].reshape(N, 1)


def kernel(x_raw, field_features, table, w):
    return _fm_forward(x_raw, field_features, table, w)


# speedup vs baseline: 1.0422x; 1.0091x over previous
import functools

import jax
import jax.numpy as jnp
from jax.experimental import pallas as pl
from jax.experimental.pallas import tpu as pltpu

_BAND = 128


def _round_up(x, m):
    return (x + m - 1) // m * m


def _fm_kernel(idx_t_ref, lhs_t_ref, out_ref, *, emb_dim, n_fields):
    _, n_blk = idx_t_ref.shape
    d_pad, _ = lhs_t_ref.shape

    idx_all = idx_t_ref[...]
    iota_b = jax.lax.broadcasted_iota(jnp.int32, (_BAND, n_blk), 0)

    counts = jnp.concatenate(
        [(iota_b == idx_all[f : f + 1, :]).astype(jnp.float32)
         for f in range(n_fields)], axis=0)

    res = jnp.dot(lhs_t_ref[...], counts,
                  preferred_element_type=jnp.float32)

    row_id = jax.lax.broadcasted_iota(jnp.int32, (d_pad, n_blk), 0)
    contrib = jnp.where(row_id < emb_dim, res * res, res)
    logit = jnp.sum(contrib, axis=0, keepdims=True)
    out_ref[...] = jax.nn.sigmoid(logit)


def _lhs_kernel(scal_ref, table_ref, w_ref, lhs_ref, fused_ref, *, emb_dim,
                n_fields, d_pad):
    V, D = table_ref.shape
    tab = table_ref[...]
    fused_ref[:V, :D] = tab
    fused_ref[:V, D + 1:] = jnp.zeros((V, 128 - D - 1), jnp.float32)
    fused_ref[:V, D:D + 1] = w_ref[...] - jnp.sum(tab * tab, axis=1,
                                                  keepdims=True)
    iota_s = jax.lax.broadcasted_iota(jnp.int32, (_BAND, 128), 0)
    for f in range(n_fields):
        off = scal_ref[0, f]
        ffv = scal_ref[1, f]
        base = pl.multiple_of((off >> 3) << 3, 8)
        chunk = fused_ref[pl.ds(base, 2 * _BAND), :]
        band = pltpu.roll(chunk, -(off & 7), axis=0)[:_BAND, :]
        band = jnp.where(iota_s < ffv, band, 0.0)
        lhs_ref[:, f * _BAND:(f + 1) * _BAND] = band.T[:d_pad, :]


def _build_lhs(table, w, offsets, field_features, d_pad):
    V, D = table.shape
    F = offsets.shape[0]
    scal = jnp.stack([offsets, jnp.asarray(field_features, jnp.int32)])
    grid_spec = pltpu.PrefetchScalarGridSpec(
        num_scalar_prefetch=1,
        grid=(1,),
        in_specs=[
            pl.BlockSpec((V, D), lambda i, s: (0, 0)),
            pl.BlockSpec((V, 1), lambda i, s: (0, 0)),
        ],
        out_specs=pl.BlockSpec((d_pad, F * _BAND), lambda i, s: (0, 0)),
        scratch_shapes=[pltpu.VMEM((_round_up(V + 2 * _BAND, 8), 128),
                                   jnp.float32)],
    )
    return pl.pallas_call(
        functools.partial(_lhs_kernel, emb_dim=D, n_fields=F, d_pad=d_pad),
        out_shape=jax.ShapeDtypeStruct((d_pad, F * _BAND), jnp.float32),
        grid_spec=grid_spec,
        compiler_params=pltpu.CompilerParams(
            dimension_semantics=("arbitrary",)),
    )(scal, table, w)


def _fm_forward(x_raw, field_features, table, w, *, n_blk=8192):
    N, F = x_raw.shape
    V, D = table.shape

    n_pad = _round_up(N, n_blk)
    d_rows = D + 1
    d_pad = _round_up(d_rows, 8)
    v_loc = F * _BAND

    x_in = x_raw.astype(jnp.int32)
    if n_pad != N:
        x_in = jnp.pad(x_in, ((0, n_pad - N), (0, 0)))
    idx_t = x_in.T

    offsets = jnp.concatenate(
        [jnp.zeros((1,), jnp.int32),
         jnp.cumsum(jnp.asarray(field_features, jnp.int32))[:-1]])
    lhs_t = _build_lhs(table, w, offsets, field_features, d_pad)

    vmem_bytes = int(
        4 * (2 * d_pad * v_loc
             + 2 * F * n_blk
             + 2 * 1 * n_blk
             + v_loc * n_blk
             + 2 * d_pad * n_blk)
        + (8 << 20))

    out = pl.pallas_call(
        functools.partial(_fm_kernel, emb_dim=D, n_fields=F),
        out_shape=jax.ShapeDtypeStruct((1, n_pad), jnp.float32),
        grid=(n_pad // n_blk,),
        in_specs=[
            pl.BlockSpec((F, n_blk), lambda i: (0, i)),
            pl.BlockSpec((d_pad, v_loc), lambda i: (0, 0)),
        ],
        out_specs=pl.BlockSpec((1, n_blk), lambda i: (0, i)),
        compiler_params=pltpu.CompilerParams(
            dimension_semantics=("parallel",),
            vmem_limit_bytes=vmem_bytes),
    )(idx_t, lhs_t)

    return out[0, :N].reshape(N, 1)


def kernel(x_raw, field_features, table, w):
    return _fm_forward(x_raw, field_features, table, w)
